# baseline (speedup 1.0000x reference)
import jax
import jax.numpy as jnp
from jax.experimental import pallas as pl
from jax.experimental.pallas import tpu as pltpu

_TILE = 2048


def _router_kernel(x_ref, w1_ref, b1_ref, wv1_ref, bv1_ref, w2_ref, b2_ref,
                   w3_ref, b3_ref, wv2_ref, bv2_ref, logits_ref, values_ref):
    x = x_ref[...]
    h = jnp.dot(x, w1_ref[...], preferred_element_type=jnp.float32)
    h = jnp.maximum(h + b1_ref[...], 0.0)
    v = jnp.dot(x, wv1_ref[...], preferred_element_type=jnp.float32)
    v = jnp.maximum(v + bv1_ref[...], 0.0)
    h2 = jnp.dot(h, w2_ref[...], preferred_element_type=jnp.float32)
    h2 = jnp.maximum(h2 + b2_ref[...], 0.0)
    logits_ref[...] = (
        jnp.dot(h2, w3_ref[...], preferred_element_type=jnp.float32)
        + b3_ref[...]
    )
    values_ref[...] = (
        jnp.dot(v, wv2_ref[...], preferred_element_type=jnp.float32)
        + bv2_ref[...]
    )


def kernel(hidden_states, W1, b1, W2, b2, W3, b3, Wv1, bv1, Wv2, bv2):
    B, S, H = hidden_states.shape
    N = B * S
    E = W3.shape[1]
    flat = hidden_states.reshape(N, H)
    logits, values = pl.pallas_call(
        _router_kernel,
        grid=(N // _TILE,),
        compiler_params=pltpu.CompilerParams(
            dimension_semantics=("parallel",),
        ),
        in_specs=[
            pl.BlockSpec((_TILE, H), lambda i: (i, 0)),
            pl.BlockSpec((H, H // 2), lambda i: (0, 0)),
            pl.BlockSpec((1, H // 2), lambda i: (0, 0)),
            pl.BlockSpec((H, H // 2), lambda i: (0, 0)),
            pl.BlockSpec((1, H // 2), lambda i: (0, 0)),
            pl.BlockSpec((H // 2, H // 4), lambda i: (0, 0)),
            pl.BlockSpec((1, H // 4), lambda i: (0, 0)),
            pl.BlockSpec((H // 4, E), lambda i: (0, 0)),
            pl.BlockSpec((1, E), lambda i: (0, 0)),
            pl.BlockSpec((H // 2, 1), lambda i: (0, 0)),
            pl.BlockSpec((1, 1), lambda i: (0, 0)),
        ],
        out_specs=[
            pl.BlockSpec((_TILE, E), lambda i: (i, 0)),
            pl.BlockSpec((_TILE, 1), lambda i: (i, 0)),
        ],
        out_shape=[
            jax.ShapeDtypeStruct((N, E), jnp.float32),
            jax.ShapeDtypeStruct((N, 1), jnp.float32),
        ],
    )(flat, W1, b1[None, :], Wv1, bv1[None, :], W2, b2[None, :],
      W3, b3[None, :], Wv2, bv2[None, :])
    return (logits.reshape(B, S, E), values.reshape(B, S, 1))


# speedup vs baseline: 1.1892x; 1.1892x over previous
import jax
import jax.numpy as jnp
from jax.experimental import pallas as pl
from jax.experimental.pallas import tpu as pltpu

_TILE = 2048


def _router_kernel(x_ref, w1_ref, b1_ref, wv1_ref, bv1_ref, w2_ref, b2_ref,
                   w3_ref, b3_ref, wv2_ref, bv2_ref, logits_ref, values_ref,
                   wcat_ref, bcat_ref):
    @pl.when(pl.program_id(0) == 0)
    def _init():
        wcat_ref[:, :384] = w1_ref[...]
        wcat_ref[:, 384:] = wv1_ref[...]
        bcat_ref[:, :384] = b1_ref[...]
        bcat_ref[:, 384:] = bv1_ref[...]

    x = x_ref[...]
    h_all = jnp.dot(x, wcat_ref[...], preferred_element_type=jnp.float32)
    h_all = jnp.maximum(h_all + bcat_ref[...], 0.0)
    h = h_all[:, :384]
    v = h_all[:, 384:]
    h2 = jnp.dot(h, w2_ref[...], preferred_element_type=jnp.float32)
    h2 = jnp.maximum(h2 + b2_ref[...], 0.0)
    logits_ref[...] = (
        jnp.dot(h2, w3_ref[...], preferred_element_type=jnp.float32)
        + b3_ref[...]
    )
    values_ref[...] = (
        jnp.dot(v, wv2_ref[...], preferred_element_type=jnp.float32)
        + bv2_ref[...]
    )


def kernel(hidden_states, W1, b1, W2, b2, W3, b3, Wv1, bv1, Wv2, bv2):
    B, S, H = hidden_states.shape
    N = B * S
    E = W3.shape[1]
    flat = hidden_states.reshape(N, H)
    logits, values = pl.pallas_call(
        _router_kernel,
        grid=(N // _TILE,),
        in_specs=[
            pl.BlockSpec((_TILE, H), lambda i: (i, 0)),
            pl.BlockSpec((H, H // 2), lambda i: (0, 0)),
            pl.BlockSpec((1, H // 2), lambda i: (0, 0)),
            pl.BlockSpec((H, H // 2), lambda i: (0, 0)),
            pl.BlockSpec((1, H // 2), lambda i: (0, 0)),
            pl.BlockSpec((H // 2, H // 4), lambda i: (0, 0)),
            pl.BlockSpec((1, H // 4), lambda i: (0, 0)),
            pl.BlockSpec((H // 4, E), lambda i: (0, 0)),
            pl.BlockSpec((1, E), lambda i: (0, 0)),
            pl.BlockSpec((H // 2, 1), lambda i: (0, 0)),
            pl.BlockSpec((1, 1), lambda i: (0, 0)),
        ],
        out_specs=[
            pl.BlockSpec((_TILE, E), lambda i: (i, 0)),
            pl.BlockSpec((_TILE, 1), lambda i: (i, 0)),
        ],
        out_shape=[
            jax.ShapeDtypeStruct((N, E), jnp.float32),
            jax.ShapeDtypeStruct((N, 1), jnp.float32),
        ],
        scratch_shapes=[
            pltpu.VMEM((H, H), jnp.float32),
            pltpu.VMEM((1, H), jnp.float32),
        ],
    )(flat, W1, b1.reshape(1, -1), Wv1, bv1.reshape(1, -1),
      W2, b2.reshape(1, -1), W3, b3.reshape(1, -1), Wv2, bv2.reshape(1, -1))
    return (logits.reshape(B, S, E), values.reshape(B, S, 1))


# speedup vs baseline: 1.5852x; 1.3330x over previous
import jax
import jax.numpy as jnp
from jax import lax
from jax.experimental import pallas as pl
from jax.experimental.pallas import tpu as pltpu

_TILE = 2048


def _router_kernel(x_ref, w1_ref, b1_ref, wv1_ref, bv1_ref, w2t_ref, b2_ref,
                   w3_ref, b3_ref, wv2_ref, bv2_ref, logits_ref, values_ref,
                   wcat_ref, bcat_ref):
    @pl.when((pl.program_id(0) == 0) & (pl.program_id(1) == 0))
    def _init():
        wcat_ref[:, :384] = w1_ref[...]
        wcat_ref[:, 384:] = wv1_ref[...]
        bcat_ref[0, :384] = b1_ref[...]
        bcat_ref[0, 384:] = bv1_ref[...]

    x = x_ref[0]
    h_all = jnp.dot(x, wcat_ref[...], preferred_element_type=jnp.float32)
    h_all = jnp.maximum(h_all + bcat_ref[...], 0.0)
    h = h_all[:, :384]
    v = h_all[:, 384:]
    h2 = lax.dot_general(h, w2t_ref[...], (((1,), (1,)), ((), ())),
                         preferred_element_type=jnp.float32)
    h2 = jnp.maximum(h2 + b2_ref[...], 0.0)
    logits_ref[0] = (
        lax.dot_general(w3_ref[...], h2, (((0,), (1,)), ((), ())),
                        preferred_element_type=jnp.float32)
        + b3_ref[...]
    )
    values_ref[0] = (
        lax.dot_general(wv2_ref[...], v, (((0,), (1,)), ((), ())),
                        preferred_element_type=jnp.float32)
        + bv2_ref[...]
    )


def kernel(hidden_states, W1, b1, W2, b2, W3, b3, Wv1, bv1, Wv2, bv2):
    B, S, H = hidden_states.shape
    E = W3.shape[1]
    logits_t, values_t = pl.pallas_call(
        _router_kernel,
        grid=(B, S // _TILE),
        in_specs=[
            pl.BlockSpec((1, _TILE, H), lambda b, i: (b, i, 0)),
            pl.BlockSpec((H, H // 2), lambda b, i: (0, 0)),
            pl.BlockSpec((H // 2,), lambda b, i: (0,)),
            pl.BlockSpec((H, H // 2), lambda b, i: (0, 0)),
            pl.BlockSpec((H // 2,), lambda b, i: (0,)),
            pl.BlockSpec((H // 4, H // 2), lambda b, i: (0, 0)),
            pl.BlockSpec((1, H // 4), lambda b, i: (0, 0)),
            pl.BlockSpec((H // 4, E), lambda b, i: (0, 0)),
            pl.BlockSpec((E, 1), lambda b, i: (0, 0)),
            pl.BlockSpec((H // 2, 1), lambda b, i: (0, 0)),
            pl.BlockSpec((1, 1), lambda b, i: (0, 0)),
        ],
        out_specs=[
            pl.BlockSpec((1, E, _TILE), lambda b, i: (b, 0, i)),
            pl.BlockSpec((1, 1, _TILE), lambda b, i: (b, 0, i)),
        ],
        out_shape=[
            jax.ShapeDtypeStruct((B, E, S), jnp.float32),
            jax.ShapeDtypeStruct((B, 1, S), jnp.float32),
        ],
        scratch_shapes=[
            pltpu.VMEM((H, H), jnp.float32),
            pltpu.VMEM((1, H), jnp.float32),
        ],
    )(hidden_states, W1, b1, Wv1, bv1, W2.T, b2.reshape(1, -1),
      W3, b3.reshape(-1, 1), Wv2, bv2.reshape(-1, 1))
    logits = jnp.transpose(logits_t, (0, 2, 1))
    values = jnp.transpose(values_t, (0, 2, 1))
    return (logits, values)
